# baseline (speedup 1.0000x reference)
import jax
import jax.numpy as jnp
from jax import lax
from jax.experimental import pallas as pl
from jax.experimental.pallas import tpu as pltpu
from jax.experimental.pallas import tpu_sc as plsc

_NUM_CLASSES = 100000
_FEAT_DIM = 64
_BATCH = 16384
_NW = 32
_BPW = _BATCH // _NW
_CHUNK = 128
_NCHUNK = _BPW // _CHUNK
_NGROUP = _BPW // 16
_MARGIN = 1.0


def _loss_body(label_hbm, feat_hbm, centers_hbm, out_hbm,
               lbl_v, rows_v, feat_v, acc_v, sem):
    wid = lax.axis_index("s") * 2 + lax.axis_index("c")
    base = wid * _BPW

    pltpu.sync_copy(label_hbm.at[pl.ds(wid * _NCHUNK, _NCHUNK)], lbl_v)
    copies = [
        pltpu.async_copy(centers_hbm.at[lbl_v.at[j]],
                         rows_v.at[pl.ds(j * _CHUNK, _CHUNK)], sem)
        for j in range(_NCHUNK)
    ]
    pltpu.sync_copy(feat_hbm.at[pl.ds(base, _BPW)], feat_v)
    for cp in copies:
        cp.wait()

    lane = lax.iota(jnp.int32, 16)

    def group(g, acc):
        rows16 = g * 16 + lane
        s = jnp.zeros((16,), jnp.float32)
        ff = jnp.zeros((16,), jnp.float32)
        dot = jnp.zeros((16,), jnp.float32)
        for k in range(_FEAT_DIM):
            col = jnp.full((16,), k, jnp.int32)
            c = plsc.load_gather(rows_v, [rows16, col])
            f = plsc.load_gather(feat_v, [rows16, col])
            s = s + c * c
            ff = ff + f * f
            dot = dot + f * c
        sc = jnp.maximum(s, jnp.float32(1e-24))
        seed = jnp.int32(0x5F3759DF) - lax.shift_right_arithmetic(
            lax.bitcast_convert_type(sc, jnp.int32), 1)
        y = lax.bitcast_convert_type(seed, jnp.float32)
        for _ in range(3):
            y = y * (jnp.float32(1.5) - jnp.float32(0.5) * sc * y * y)
        d = ff - 2.0 * (dot * y) + s * (y * y) - _MARGIN
        return acc + jnp.maximum(jnp.exp(d) - 1.0, 0.0)

    acc = lax.fori_loop(0, _NGROUP, group, jnp.zeros((16,), jnp.float32))
    acc_v[...] = acc
    pltpu.sync_copy(acc_v, out_hbm.at[wid])


_sc_loss = pl.kernel(
    _loss_body,
    mesh=plsc.VectorSubcoreMesh(core_axis_name="c", subcore_axis_name="s"),
    compiler_params=pltpu.CompilerParams(
        needs_layout_passes=False, use_tc_tiling_on_sc=False),
    out_type=jax.ShapeDtypeStruct((_NW, 16), jnp.float32),
    scratch_types=[
        pltpu.VMEM((_NCHUNK, _CHUNK), jnp.int32),
        pltpu.VMEM((_BPW, _FEAT_DIM), jnp.float32),
        pltpu.VMEM((_BPW, _FEAT_DIM), jnp.float32),
        pltpu.VMEM((16,), jnp.float32),
        pltpu.SemaphoreType.DMA,
    ],
)


def kernel(label, feat, centers):
    lbl = label.astype(jnp.int32).reshape(_BATCH // _CHUNK, _CHUNK)
    partials = _sc_loss(lbl, feat, centers)
    return jnp.sum(partials) / 2.0 / _BATCH


# speedup vs baseline: 1.1018x; 1.1018x over previous
import jax
import jax.numpy as jnp
from jax import lax
from jax.experimental import pallas as pl
from jax.experimental.pallas import tpu as pltpu
from jax.experimental.pallas import tpu_sc as plsc

_NUM_CLASSES = 100000
_FEAT_DIM = 64
_BATCH = 16384
_NW = 32
_BPW = _BATCH // _NW
_CHUNK = 128
_NCHUNK = _BPW // _CHUNK
_GPC = _CHUNK // 16
_MARGIN = 1.0


def _loss_body(label_hbm, featT_hbm, pairs_hbm, out_hbm,
               lbl_v, idx2_v, rows_v, featT_v, acc_v, sem):
    wid = lax.axis_index("s") * 2 + lax.axis_index("c")
    base = wid * _BPW

    pltpu.sync_copy(label_hbm.at[pl.ds(base, _BPW)], lbl_v)
    for t in range(_BPW // 16):
        idx2_v[pl.ds(t * 16, 16)] = lax.shift_right_logical(
            lbl_v[pl.ds(t * 16, 16)], 1)
    copies = [
        pltpu.async_copy(pairs_hbm.at[idx2_v.at[pl.ds(j * _CHUNK, _CHUNK)]],
                         rows_v.at[pl.ds(j * _CHUNK, _CHUNK)], sem)
        for j in range(_NCHUNK)
    ]
    pltpu.sync_copy(featT_hbm.at[:, pl.ds(base, _BPW)], featT_v)

    lane = lax.iota(jnp.int32, 16)
    acc0 = jnp.zeros((16,), jnp.float32)

    def make_group(j):
        def group(gi, acc):
            g16 = j * _CHUNK + gi * 16
            rows16 = g16 + lane
            lbl16 = lbl_v[pl.ds(g16, 16)]
            par64 = lax.shift_left(jnp.bitwise_and(lbl16, 1), 6)
            s = jnp.zeros((16,), jnp.float32)
            ff = jnp.zeros((16,), jnp.float32)
            dot = jnp.zeros((16,), jnp.float32)
            for k in range(_FEAT_DIM):
                col = par64 + k
                c = plsc.load_gather(rows_v, [rows16, col])
                f = featT_v[k, pl.ds(g16, 16)]
                s = s + c * c
                ff = ff + f * f
                dot = dot + f * c
            sc = jnp.maximum(s, jnp.float32(1e-24))
            seed = jnp.int32(0x5F3759DF) - lax.shift_right_arithmetic(
                lax.bitcast_convert_type(sc, jnp.int32), 1)
            y = lax.bitcast_convert_type(seed, jnp.float32)
            for _ in range(3):
                y = y * (jnp.float32(1.5) - jnp.float32(0.5) * sc * y * y)
            d = ff - 2.0 * (dot * y) + s * (y * y) - _MARGIN
            return acc + jnp.maximum(jnp.exp(d) - 1.0, 0.0)
        return group

    acc = acc0
    for j in range(_NCHUNK):
        copies[j].wait()
        acc = lax.fori_loop(0, _GPC, make_group(j), acc)

    acc_v[...] = acc
    pltpu.sync_copy(acc_v, out_hbm.at[pl.ds(wid * 16, 16)])


_sc_loss = pl.kernel(
    _loss_body,
    mesh=plsc.VectorSubcoreMesh(core_axis_name="c", subcore_axis_name="s"),
    compiler_params=pltpu.CompilerParams(needs_layout_passes=False),
    out_type=jax.ShapeDtypeStruct((_NW * 16,), jnp.float32),
    scratch_types=[
        pltpu.VMEM((_BPW,), jnp.int32),
        pltpu.VMEM((_BPW,), jnp.int32),
        pltpu.VMEM((_BPW, 2 * _FEAT_DIM), jnp.float32),
        pltpu.VMEM((_FEAT_DIM, _BPW), jnp.float32),
        pltpu.VMEM((16,), jnp.float32),
        pltpu.SemaphoreType.DMA,
    ],
)


def kernel(label, feat, centers):
    pairs = centers.reshape(_NUM_CLASSES // 2, 2 * _FEAT_DIM)
    partials = _sc_loss(label.astype(jnp.int32), feat.T, pairs)
    return jnp.sum(partials) / 2.0 / _BATCH
